# baseline (speedup 1.0000x reference)
import jax
import jax.numpy as jnp
from jax.experimental import pallas as pl
from jax.experimental.pallas import tpu as pltpu

C = 128
D = 131072
CHUNK = 4096
NCHUNK = D // CHUNK


def _body(x_ref, out_ref, g_ref, sq_ref, w_ref):
    p = pl.program_id(0)
    i = pl.program_id(1)

    @pl.when(p == 0)
    def _phase0():
        x = x_ref[...]
        xb = x.astype(jnp.bfloat16)
        g = jax.lax.dot_general(
            xb,
            xb,
            (((0,), (0,)), ((), ())),
            preferred_element_type=jnp.float32,
        )
        sq = jnp.sum(x * x, axis=0, keepdims=True)

        @pl.when(i == 0)
        def _init():
            g_ref[...] = g
            sq_ref[...] = sq

        @pl.when(i > 0)
        def _acc():
            g_ref[...] = g_ref[...] + g
            sq_ref[...] = sq_ref[...] + sq

        @pl.when(i == NCHUNK - 1)
        def _finalize():
            n = C
            G = g_ref[...]
            sq_row = sq_ref[...]
            rows = jax.lax.broadcasted_iota(jnp.int32, (C, C), 0)
            cols = jax.lax.broadcasted_iota(jnp.int32, (C, C), 1)
            eye = rows == cols
            sq_col = jnp.sum(
                jnp.where(eye, jnp.broadcast_to(sq_row, (C, C)), 0.0),
                axis=1,
                keepdims=True,
            )
            d2 = sq_col + sq_row - 2.0 * G
            d2 = jnp.maximum(d2, 0.0)
            cs = jnp.where(d2 > 0.0, jnp.sqrt(jnp.where(d2 > 0.0, d2, 1.0)), 0.0)

            def rank_step(k, S):
                colk = jnp.sum(
                    jnp.where(cols == k, cs, 0.0), axis=1, keepdims=True
                )
                less = (colk < cs).astype(jnp.float32)
                tie = jnp.logical_and(colk == cs, k < cols).astype(jnp.float32)
                return S + jnp.sum(less + tie, axis=0, keepdims=True)

            S = jax.lax.fori_loop(
                0, C, rank_step, jnp.zeros((1, C), jnp.float32)
            )
            wrow = (1.0 - 2.0 * (S / n) / (n - 1)) / n
            csb = cs.astype(jnp.bfloat16).astype(jnp.float32)
            wb = wrow.astype(jnp.bfloat16).astype(jnp.float32)
            vals = jnp.sum(csb * wb, axis=1, keepdims=True)
            wv = -1.0 * vals + 6.0
            wv = wv / jnp.max(wv)
            wv = jnp.where(wv == 1.0, 0.99, wv)
            wv = jnp.where(wv == 0.0, 0.01, wv)
            wv = jnp.log(wv / (1.0 - wv)) + 0.5
            wv = jnp.where((jnp.isinf(wv).astype(wv.dtype) + wv) > 1.0, 1.0, wv)
            wv = jnp.where(wv < 0.0, 0.0, wv)
            wv = wv / jnp.sum(wv)
            w_ref[...] = jnp.sum(
                jnp.where(eye, jnp.broadcast_to(wv, (C, C)), 0.0),
                axis=0,
                keepdims=True,
            )

    @pl.when(p == 1)
    def _phase1():
        x = x_ref[...]
        out_ref[:, pl.ds(i * CHUNK, CHUNK)] = jax.lax.dot_general(
            w_ref[...],
            x,
            (((1,), (1,)), ((), ())),
            preferred_element_type=jnp.float32,
            precision=jax.lax.Precision.HIGHEST,
        )


def kernel(input):
    x = jnp.reshape(input, (D, C))
    out_row = pl.pallas_call(
        _body,
        grid=(2, NCHUNK),
        in_specs=[pl.BlockSpec((CHUNK, C), lambda p, i: (i, 0))],
        out_specs=pl.BlockSpec((1, D), lambda p, i: (0, 0)),
        out_shape=jax.ShapeDtypeStruct((1, D), jnp.float32),
        scratch_shapes=[
            pltpu.VMEM((C, C), jnp.float32),
            pltpu.VMEM((1, C), jnp.float32),
            pltpu.VMEM((1, C), jnp.float32),
        ],
    )(x)
    return jnp.reshape(out_row, (D, 1))


# speedup vs baseline: 1.7943x; 1.7943x over previous
import jax
import jax.numpy as jnp
from jax.experimental import pallas as pl
from jax.experimental.pallas import tpu as pltpu

C = 128
D = 131072
CHUNK = 4096
NCHUNK = D // CHUNK
OCHUNK = 16384
NOCHUNK = D // OCHUNK


def _body(x_ref, out_ref, g_ref, sq_ref, cache_ref):
    i = pl.program_id(0)

    x = x_ref[...]
    xb = x.astype(jnp.bfloat16)
    cache_ref[pl.ds(i * CHUNK, CHUNK), :] = xb
    g = jax.lax.dot_general(
        xb,
        xb,
        (((0,), (0,)), ((), ())),
        preferred_element_type=jnp.float32,
    )
    sq = jnp.sum(x * x, axis=0, keepdims=True)

    @pl.when(i == 0)
    def _init():
        g_ref[...] = g
        sq_ref[...] = sq

    @pl.when(i > 0)
    def _acc():
        g_ref[...] = g_ref[...] + g
        sq_ref[...] = sq_ref[...] + sq

    @pl.when(i == NCHUNK - 1)
    def _finalize():
        n = C
        G = g_ref[...]
        sq_row = sq_ref[...]
        rows = jax.lax.broadcasted_iota(jnp.int32, (C, C), 0)
        cols = jax.lax.broadcasted_iota(jnp.int32, (C, C), 1)
        eye = rows == cols
        sq_col = jnp.sum(
            jnp.where(eye, jnp.broadcast_to(sq_row, (C, C)), 0.0),
            axis=1,
            keepdims=True,
        )
        d2 = sq_col + sq_row - 2.0 * G
        d2 = jnp.maximum(d2, 0.0)
        cs = jnp.where(d2 > 0.0, jnp.sqrt(jnp.where(d2 > 0.0, d2, 1.0)), 0.0)

        def rank_step(k, S):
            colk = jnp.sum(
                jnp.where(cols == k, cs, 0.0), axis=1, keepdims=True
            )
            less = (colk < cs).astype(jnp.float32)
            tie = jnp.logical_and(colk == cs, k < cols).astype(jnp.float32)
            return S + jnp.sum(less + tie, axis=0, keepdims=True)

        S = jax.lax.fori_loop(0, C, rank_step, jnp.zeros((1, C), jnp.float32))
        wrow = (1.0 - 2.0 * (S / n) / (n - 1)) / n
        csb = cs.astype(jnp.bfloat16).astype(jnp.float32)
        wb = wrow.astype(jnp.bfloat16).astype(jnp.float32)
        vals = jnp.sum(csb * wb, axis=1, keepdims=True)
        wv = -1.0 * vals + 6.0
        wv = wv / jnp.max(wv)
        wv = jnp.where(wv == 1.0, 0.99, wv)
        wv = jnp.where(wv == 0.0, 0.01, wv)
        wv = jnp.log(wv / (1.0 - wv)) + 0.5
        wv = jnp.where((jnp.isinf(wv).astype(wv.dtype) + wv) > 1.0, 1.0, wv)
        wv = jnp.where(wv < 0.0, 0.0, wv)
        wv = wv / jnp.sum(wv)
        wfin = jnp.sum(
            jnp.where(eye, jnp.broadcast_to(wv, (C, C)), 0.0),
            axis=0,
            keepdims=True,
        ).astype(jnp.bfloat16)

        for j in range(NOCHUNK):
            xc = cache_ref[pl.ds(j * OCHUNK, OCHUNK), :]
            out_ref[:, pl.ds(j * OCHUNK, OCHUNK)] = jax.lax.dot_general(
                wfin,
                xc,
                (((1,), (1,)), ((), ())),
                preferred_element_type=jnp.float32,
            )


def kernel(input):
    x = jnp.reshape(input, (D, C))
    out_row = pl.pallas_call(
        _body,
        grid=(NCHUNK,),
        in_specs=[pl.BlockSpec((CHUNK, C), lambda i: (i, 0))],
        out_specs=pl.BlockSpec((1, D), lambda i: (0, 0)),
        out_shape=jax.ShapeDtypeStruct((1, D), jnp.float32),
        scratch_shapes=[
            pltpu.VMEM((C, C), jnp.float32),
            pltpu.VMEM((1, C), jnp.float32),
            pltpu.VMEM((D, C), jnp.bfloat16),
        ],
    )(x)
    return jnp.reshape(out_row, (D, 1))


# speedup vs baseline: 2.0571x; 1.1464x over previous
import jax
import jax.numpy as jnp
from jax.experimental import pallas as pl
from jax.experimental.pallas import tpu as pltpu

C = 128
D = 131072
CHUNK = 8192
NCHUNK = D // CHUNK
OCHUNK = 16384
NOCHUNK = D // OCHUNK


def _body(x_ref, out_ref, g_ref, sq_ref, cache_ref):
    i = pl.program_id(0)

    x = x_ref[...]
    xb = x.astype(jnp.bfloat16)
    cache_ref[pl.ds(i * CHUNK, CHUNK), :] = xb
    g = jax.lax.dot_general(
        xb,
        xb,
        (((0,), (0,)), ((), ())),
        preferred_element_type=jnp.float32,
    )
    sq = jnp.sum(x * x, axis=0, keepdims=True)

    @pl.when(i == 0)
    def _init():
        g_ref[...] = g
        sq_ref[...] = sq

    @pl.when(i > 0)
    def _acc():
        g_ref[...] = g_ref[...] + g
        sq_ref[...] = sq_ref[...] + sq

    @pl.when(i == NCHUNK - 1)
    def _finalize():
        n = C
        G = g_ref[...]
        sq_row = sq_ref[...]
        rows = jax.lax.broadcasted_iota(jnp.int32, (C, C), 0)
        cols = jax.lax.broadcasted_iota(jnp.int32, (C, C), 1)
        eye = rows == cols
        sq_col = jnp.sum(
            jnp.where(eye, jnp.broadcast_to(sq_row, (C, C)), 0.0),
            axis=1,
            keepdims=True,
        )
        d2 = sq_col + sq_row - 2.0 * G
        d2 = jnp.maximum(d2, 0.0)
        cs = jnp.where(d2 > 0.0, jnp.sqrt(jnp.where(d2 > 0.0, d2, 1.0)), 0.0)

        def rank_step(k, S):
            colk = jnp.sum(
                jnp.where(cols == k, cs, 0.0), axis=1, keepdims=True
            )
            less = (colk < cs).astype(jnp.float32)
            tie = jnp.logical_and(colk == cs, k < cols).astype(jnp.float32)
            return S + jnp.sum(less + tie, axis=0, keepdims=True)

        S = jax.lax.fori_loop(0, C, rank_step, jnp.zeros((1, C), jnp.float32))
        wrow = (1.0 - 2.0 * (S / n) / (n - 1)) / n
        csb = cs.astype(jnp.bfloat16).astype(jnp.float32)
        wb = wrow.astype(jnp.bfloat16).astype(jnp.float32)
        vals = jnp.sum(csb * wb, axis=1, keepdims=True)
        wv = -1.0 * vals + 6.0
        wv = wv / jnp.max(wv)
        wv = jnp.where(wv == 1.0, 0.99, wv)
        wv = jnp.where(wv == 0.0, 0.01, wv)
        wv = jnp.log(wv / (1.0 - wv)) + 0.5
        wv = jnp.where((jnp.isinf(wv).astype(wv.dtype) + wv) > 1.0, 1.0, wv)
        wv = jnp.where(wv < 0.0, 0.0, wv)
        wv = wv / jnp.sum(wv)
        wfin = jnp.sum(
            jnp.where(eye, jnp.broadcast_to(wv, (C, C)), 0.0),
            axis=0,
            keepdims=True,
        ).astype(jnp.bfloat16)

        for j in range(NOCHUNK):
            xc = cache_ref[pl.ds(j * OCHUNK, OCHUNK), :]
            out_ref[:, pl.ds(j * OCHUNK, OCHUNK)] = jax.lax.dot_general(
                wfin,
                xc,
                (((1,), (1,)), ((), ())),
                preferred_element_type=jnp.float32,
            )


def kernel(input):
    x = jnp.reshape(input, (D, C))
    out_row = pl.pallas_call(
        _body,
        grid=(NCHUNK,),
        in_specs=[pl.BlockSpec((CHUNK, C), lambda i: (i, 0))],
        out_specs=pl.BlockSpec((1, D), lambda i: (0, 0)),
        out_shape=jax.ShapeDtypeStruct((1, D), jnp.float32),
        scratch_shapes=[
            pltpu.VMEM((C, C), jnp.float32),
            pltpu.VMEM((1, C), jnp.float32),
            pltpu.VMEM((D, C), jnp.bfloat16),
        ],
    )(x)
    return jnp.reshape(out_row, (D, 1))


# speedup vs baseline: 2.4354x; 1.1839x over previous
import jax
import jax.numpy as jnp
from jax.experimental import pallas as pl
from jax.experimental.pallas import tpu as pltpu

C = 128
D = 131072
CHUNK = 8192
NCHUNK = D // CHUNK
OCHUNK = 16384
NOCHUNK = D // OCHUNK


def _body(x_ref, out_ref, g_ref, sq_ref, cache_ref):
    i = pl.program_id(0)

    x = x_ref[...]
    xb = x.astype(jnp.bfloat16)
    cache_ref[pl.ds(i * CHUNK, CHUNK), :] = xb
    g = jax.lax.dot_general(
        xb,
        xb,
        (((0,), (0,)), ((), ())),
        preferred_element_type=jnp.float32,
    )
    sq = jnp.sum(x * x, axis=0, keepdims=True)

    @pl.when(i == 0)
    def _init():
        g_ref[...] = g
        sq_ref[...] = sq

    @pl.when(i > 0)
    def _acc():
        g_ref[...] = g_ref[...] + g
        sq_ref[...] = sq_ref[...] + sq

    @pl.when(i == NCHUNK - 1)
    def _finalize():
        n = C
        G = g_ref[...]
        sq_row = sq_ref[...]
        rows = jax.lax.broadcasted_iota(jnp.int32, (C, C), 0)
        cols = jax.lax.broadcasted_iota(jnp.int32, (C, C), 1)
        eye = rows == cols
        sq_col = jnp.sum(
            jnp.where(eye, jnp.broadcast_to(sq_row, (C, C)), 0.0),
            axis=1,
            keepdims=True,
        )
        d2 = sq_col + sq_row - 2.0 * G
        d2 = jnp.maximum(d2, 0.0)
        cs = jnp.where(d2 > 0.0, jnp.sqrt(jnp.where(d2 > 0.0, d2, 1.0)), 0.0)

        acc = jnp.zeros((C, C), jnp.float32)
        for k in range(C):
            colk = cs[:, k : k + 1]
            hit = jnp.logical_or(
                colk < cs, jnp.logical_and(colk == cs, cols > k)
            )
            acc = acc + hit.astype(jnp.float32)
        S = jnp.sum(acc, axis=0, keepdims=True)
        wrow = (1.0 - 2.0 * (S / n) / (n - 1)) / n
        csb = cs.astype(jnp.bfloat16).astype(jnp.float32)
        wb = wrow.astype(jnp.bfloat16).astype(jnp.float32)
        vals = jnp.sum(csb * wb, axis=1, keepdims=True)
        wv = -1.0 * vals + 6.0
        wv = wv / jnp.max(wv)
        wv = jnp.where(wv == 1.0, 0.99, wv)
        wv = jnp.where(wv == 0.0, 0.01, wv)
        wv = jnp.log(wv / (1.0 - wv)) + 0.5
        wv = jnp.where((jnp.isinf(wv).astype(wv.dtype) + wv) > 1.0, 1.0, wv)
        wv = jnp.where(wv < 0.0, 0.0, wv)
        wv = wv / jnp.sum(wv)
        wfin = jnp.sum(
            jnp.where(eye, jnp.broadcast_to(wv, (C, C)), 0.0),
            axis=0,
            keepdims=True,
        ).astype(jnp.bfloat16)

        for j in range(NOCHUNK):
            xc = cache_ref[pl.ds(j * OCHUNK, OCHUNK), :]
            out_ref[:, pl.ds(j * OCHUNK, OCHUNK)] = jax.lax.dot_general(
                wfin,
                xc,
                (((1,), (1,)), ((), ())),
                preferred_element_type=jnp.float32,
            )


def kernel(input):
    x = jnp.reshape(input, (D, C))
    out_row = pl.pallas_call(
        _body,
        grid=(NCHUNK,),
        in_specs=[pl.BlockSpec((CHUNK, C), lambda i: (i, 0))],
        out_specs=pl.BlockSpec((1, D), lambda i: (0, 0)),
        out_shape=jax.ShapeDtypeStruct((1, D), jnp.float32),
        scratch_shapes=[
            pltpu.VMEM((C, C), jnp.float32),
            pltpu.VMEM((1, C), jnp.float32),
            pltpu.VMEM((D, C), jnp.bfloat16),
        ],
    )(x)
    return jnp.reshape(out_row, (D, 1))


# speedup vs baseline: 2.4365x; 1.0005x over previous
import jax
import jax.numpy as jnp
from jax.experimental import pallas as pl
from jax.experimental.pallas import tpu as pltpu

C = 128
D = 131072
CHUNK = 8192
NCHUNK = D // CHUNK
OCHUNK = 32768
NOCHUNK = D // OCHUNK


def _body(x_ref, out_ref, g_ref, sq_ref, cache_ref):
    i = pl.program_id(0)

    x = x_ref[...]
    xb = x.astype(jnp.bfloat16)
    cache_ref[pl.ds(i * CHUNK, CHUNK), :] = xb
    g = jax.lax.dot_general(
        xb,
        xb,
        (((0,), (0,)), ((), ())),
        preferred_element_type=jnp.float32,
    )
    sq = jnp.sum(x * x, axis=0, keepdims=True)

    @pl.when(i == 0)
    def _init():
        g_ref[...] = g
        sq_ref[...] = sq

    @pl.when(i > 0)
    def _acc():
        g_ref[...] = g_ref[...] + g
        sq_ref[...] = sq_ref[...] + sq

    @pl.when(i == NCHUNK - 1)
    def _finalize():
        n = C
        G = g_ref[...]
        sq_row = sq_ref[...]
        rows = jax.lax.broadcasted_iota(jnp.int32, (C, C), 0)
        cols = jax.lax.broadcasted_iota(jnp.int32, (C, C), 1)
        eye = rows == cols
        sq_col = jnp.sum(
            jnp.where(eye, jnp.broadcast_to(sq_row, (C, C)), 0.0),
            axis=1,
            keepdims=True,
        )
        d2 = sq_col + sq_row - 2.0 * G
        d2 = jnp.maximum(d2, 0.0)
        cs = jnp.where(d2 > 0.0, jnp.sqrt(jnp.where(d2 > 0.0, d2, 1.0)), 0.0)

        acc = jnp.zeros((C, C), jnp.float32)
        for k in range(C):
            colk = cs[:, k : k + 1]
            hit = jnp.logical_or(
                colk < cs, jnp.logical_and(colk == cs, cols > k)
            )
            acc = acc + hit.astype(jnp.float32)
        S = jnp.sum(acc, axis=0, keepdims=True)
        wrow = (1.0 - 2.0 * (S / n) / (n - 1)) / n
        csb = cs.astype(jnp.bfloat16).astype(jnp.float32)
        wb = wrow.astype(jnp.bfloat16).astype(jnp.float32)
        vals = jnp.sum(csb * wb, axis=1, keepdims=True)
        wv = -1.0 * vals + 6.0
        wv = wv / jnp.max(wv)
        wv = jnp.where(wv == 1.0, 0.99, wv)
        wv = jnp.where(wv == 0.0, 0.01, wv)
        wv = jnp.log(wv / (1.0 - wv)) + 0.5
        wv = jnp.where((jnp.isinf(wv).astype(wv.dtype) + wv) > 1.0, 1.0, wv)
        wv = jnp.where(wv < 0.0, 0.0, wv)
        wv = wv / jnp.sum(wv)
        wfin = jnp.sum(
            jnp.where(eye, jnp.broadcast_to(wv, (C, C)), 0.0),
            axis=0,
            keepdims=True,
        ).astype(jnp.bfloat16)

        for j in range(NOCHUNK):
            xc = cache_ref[pl.ds(j * OCHUNK, OCHUNK), :]
            out_ref[:, pl.ds(j * OCHUNK, OCHUNK)] = jax.lax.dot_general(
                wfin,
                xc,
                (((1,), (1,)), ((), ())),
                preferred_element_type=jnp.float32,
            )


def kernel(input):
    x = jnp.reshape(input, (D, C))
    out_row = pl.pallas_call(
        _body,
        grid=(NCHUNK,),
        in_specs=[pl.BlockSpec((CHUNK, C), lambda i: (i, 0))],
        out_specs=pl.BlockSpec((1, D), lambda i: (0, 0)),
        out_shape=jax.ShapeDtypeStruct((1, D), jnp.float32),
        scratch_shapes=[
            pltpu.VMEM((C, C), jnp.float32),
            pltpu.VMEM((1, C), jnp.float32),
            pltpu.VMEM((D, C), jnp.bfloat16),
        ],
    )(x)
    return jnp.reshape(out_row, (D, 1))
